# Initial kernel scaffold; baseline (speedup 1.0000x reference)
import jax
import jax.numpy as jnp
from jax.experimental import pallas as pl


def kernel(x, edge_index, W1, b1, W2, b2, W3, b3):
    raise NotImplementedError("write your pallas kernel here")


# speedup vs baseline: 4.8225x; 4.8225x over previous
import functools

import jax
import jax.numpy as jnp
from jax import lax
from jax.experimental import pallas as pl
from jax.experimental.pallas import tpu as pltpu
from jax.experimental.pallas import tpu_sc as plsc

N_NODES = 10000
N_EDGES = 320000
F_IN = 1433
HID = 128
N_CLS = 7

NC = 2
NS = 16
CHUNK = 128
ACC_ROWS = 10112
JUNK_ROW = 10048
E_PAD = 327680
EDGES_PER_CORE = E_PAD // NC
EDGES_PER_TILE = EDGES_PER_CORE // NS
N_CHUNKS = EDGES_PER_TILE // CHUNK
INIT_ROWS = ACC_ROWS // NS

K_PAD = 1536


def _sc_mesh():
    return plsc.VectorSubcoreMesh(core_axis_name="c", subcore_axis_name="s")


def _deg_kernel(col_hbm, ones_hbm, zeros_hbm, out_hbm, col_v, ones_v, acc_sh):
    c = lax.axis_index("c")
    s = lax.axis_index("s")
    pltpu.sync_copy(zeros_hbm, acc_sh.at[pl.ds(s * INIT_ROWS, INIT_ROWS)])
    pltpu.sync_copy(ones_hbm, ones_v)
    plsc.subcore_barrier()
    base = c * EDGES_PER_CORE + s * EDGES_PER_TILE

    @pl.loop(0, N_CHUNKS)
    def _(i):
        pltpu.sync_copy(col_hbm.at[pl.ds(base + i * CHUNK, CHUNK)], col_v)
        pltpu.sync_copy(ones_v, acc_sh.at[col_v], add=True)

    plsc.subcore_barrier()
    sl = pl.ds(s * INIT_ROWS, INIT_ROWS)
    pltpu.sync_copy(acc_sh.at[sl], out_hbm.at[c].at[sl])


def _degrees(col_pad, ones16, zeros16):
    kfn = pl.kernel(
        _deg_kernel,
        out_type=jax.ShapeDtypeStruct((NC, ACC_ROWS, 16), jnp.float32),
        mesh=_sc_mesh(),
        scratch_types=[
            pltpu.VMEM((CHUNK,), jnp.int32),
            pltpu.VMEM((CHUNK, 16), jnp.float32),
            pltpu.VMEM_SHARED((ACC_ROWS, 16), jnp.float32),
        ],
    )
    return kfn(col_pad, ones16, zeros16)


def _agg_kernel(hs_hbm, row_hbm, col_hbm, zeros_hbm, out_hbm,
                row_v, col_v, rows_v, acc_sh):
    c = lax.axis_index("c")
    s = lax.axis_index("s")
    pltpu.sync_copy(zeros_hbm, acc_sh.at[pl.ds(s * INIT_ROWS, INIT_ROWS)])
    plsc.subcore_barrier()
    base = c * EDGES_PER_CORE + s * EDGES_PER_TILE

    @pl.loop(0, N_CHUNKS)
    def _(i):
        off = base + i * CHUNK
        pltpu.sync_copy(row_hbm.at[pl.ds(off, CHUNK)], row_v)
        pltpu.sync_copy(col_hbm.at[pl.ds(off, CHUNK)], col_v)
        pltpu.sync_copy(hs_hbm.at[row_v], rows_v)
        pltpu.sync_copy(rows_v, acc_sh.at[col_v], add=True)

    plsc.subcore_barrier()
    sl = pl.ds(s * INIT_ROWS, INIT_ROWS)
    pltpu.sync_copy(acc_sh.at[sl], out_hbm.at[c].at[sl])


def _aggregate(hs, row_pad, col_pad, zeros, width):
    kfn = pl.kernel(
        _agg_kernel,
        out_type=jax.ShapeDtypeStruct((NC, ACC_ROWS, width), jnp.float32),
        mesh=_sc_mesh(),
        scratch_types=[
            pltpu.VMEM((CHUNK,), jnp.int32),
            pltpu.VMEM((CHUNK,), jnp.int32),
            pltpu.VMEM((CHUNK, width), jnp.float32),
            pltpu.VMEM_SHARED((ACC_ROWS, width), jnp.float32),
        ],
    )
    return kfn(hs, row_pad, col_pad, zeros)


_BM = 1000


def _mm1_body(x_ref, w_ref, o_ref):
    o_ref[...] = jnp.dot(x_ref[...], w_ref[...],
                         preferred_element_type=jnp.float32,
                         precision=lax.Precision.HIGHEST)


def _mm1(x_pad, w_pad):
    return pl.pallas_call(
        _mm1_body,
        grid=(N_NODES // _BM,),
        in_specs=[pl.BlockSpec((_BM, K_PAD), lambda i: (i, 0)),
                  pl.BlockSpec((K_PAD, HID), lambda i: (0, 0))],
        out_specs=pl.BlockSpec((_BM, HID), lambda i: (i, 0)),
        out_shape=jax.ShapeDtypeStruct((N_NODES, HID), jnp.float32),
    )(x_pad, w_pad)


def _scale1_body(deg_ref, h_ref, dis_ref, hs_ref):
    deg = deg_ref[0] + deg_ref[1] + 1.0
    dis = lax.rsqrt(deg)
    dis_ref[...] = dis
    hs_ref[...] = h_ref[...] * dis[:, :1]


def _scale1(deg, h1):
    return pl.pallas_call(
        _scale1_body,
        grid=(N_NODES // _BM,),
        in_specs=[pl.BlockSpec((NC, _BM, 16), lambda i: (0, i, 0)),
                  pl.BlockSpec((_BM, HID), lambda i: (i, 0))],
        out_specs=[pl.BlockSpec((_BM, 16), lambda i: (i, 0)),
                   pl.BlockSpec((_BM, HID), lambda i: (i, 0))],
        out_shape=[jax.ShapeDtypeStruct((N_NODES, 16), jnp.float32),
                   jax.ShapeDtypeStruct((N_NODES, HID), jnp.float32)],
    )(deg, h1)


def _layer_body(p_ref, hs_ref, dis_ref, b_ref, w_ref, o_ref):
    dis = dis_ref[:, :1]
    agg = p_ref[0] + p_ref[1] + hs_ref[...]
    z = jnp.maximum(agg * dis + b_ref[...], 0.0)
    h = jnp.dot(z, w_ref[...], preferred_element_type=jnp.float32,
                precision=lax.Precision.HIGHEST)
    o_ref[...] = h * dis


def _layer(partials, hs_prev, dis16, b_prev, w_next):
    return pl.pallas_call(
        _layer_body,
        grid=(N_NODES // _BM,),
        in_specs=[pl.BlockSpec((NC, _BM, HID), lambda i: (0, i, 0)),
                  pl.BlockSpec((_BM, HID), lambda i: (i, 0)),
                  pl.BlockSpec((_BM, 16), lambda i: (i, 0)),
                  pl.BlockSpec((1, HID), lambda i: (0, 0)),
                  pl.BlockSpec((HID, HID), lambda i: (0, 0))],
        out_specs=pl.BlockSpec((_BM, HID), lambda i: (i, 0)),
        out_shape=jax.ShapeDtypeStruct((N_NODES, HID), jnp.float32),
    )(partials, hs_prev, dis16, b_prev, w_next)


def _prescale3_body(p_ref, hs_ref, dis_ref, b_ref, o_ref):
    dis = dis_ref[:, :1]
    agg = p_ref[0] + p_ref[1] + hs_ref[...]
    o_ref[...] = jnp.maximum(agg * dis + b_ref[...], 0.0) * dis


def _prescale3(partials, hs_prev, dis16, b_prev):
    return pl.pallas_call(
        _prescale3_body,
        grid=(N_NODES // _BM,),
        in_specs=[pl.BlockSpec((NC, _BM, HID), lambda i: (0, i, 0)),
                  pl.BlockSpec((_BM, HID), lambda i: (i, 0)),
                  pl.BlockSpec((_BM, 16), lambda i: (i, 0)),
                  pl.BlockSpec((1, HID), lambda i: (0, 0))],
        out_specs=pl.BlockSpec((_BM, HID), lambda i: (i, 0)),
        out_shape=jax.ShapeDtypeStruct((N_NODES, HID), jnp.float32),
    )(partials, hs_prev, dis16, b_prev)


def _final_body(p_ref, q_ref, dis_ref, w_ref, b_ref, o_ref):
    t = (p_ref[0] + p_ref[1] + q_ref[...]) * dis_ref[:, :1]
    o_ref[...] = jnp.dot(t, w_ref[...], preferred_element_type=jnp.float32,
                         precision=lax.Precision.HIGHEST) + b_ref[...]


def _final(partials, q2, dis16, w3_pad, b3_pad):
    return pl.pallas_call(
        _final_body,
        grid=(N_NODES // _BM,),
        in_specs=[pl.BlockSpec((NC, _BM, HID), lambda i: (0, i, 0)),
                  pl.BlockSpec((_BM, HID), lambda i: (i, 0)),
                  pl.BlockSpec((_BM, 16), lambda i: (i, 0)),
                  pl.BlockSpec((HID, 16), lambda i: (0, 0)),
                  pl.BlockSpec((1, 16), lambda i: (0, 0))],
        out_specs=pl.BlockSpec((_BM, 16), lambda i: (i, 0)),
        out_shape=jax.ShapeDtypeStruct((N_NODES, 16), jnp.float32),
    )(partials, q2, dis16, w3_pad, b3_pad)


def kernel(x, edge_index, W1, b1, W2, b2, W3, b3):
    f32 = jnp.float32
    row = edge_index[0].astype(jnp.int32)
    col = edge_index[1].astype(jnp.int32)
    pad = E_PAD - N_EDGES
    row_pad = jnp.concatenate([row, jnp.zeros((pad,), jnp.int32)])
    col_pad = jnp.concatenate([col, jnp.full((pad,), JUNK_ROW, jnp.int32)])

    ones16 = jnp.ones((CHUNK, 16), f32)
    zeros16 = jnp.zeros((INIT_ROWS, 16), f32)
    zeros128 = jnp.zeros((INIT_ROWS, HID), f32)

    x_pad = jnp.pad(x, ((0, 0), (0, K_PAD - F_IN)))
    w1_pad = jnp.pad(W1, ((0, K_PAD - F_IN), (0, 0)))
    w3_pad = jnp.pad(W3, ((0, 0), (0, 16 - N_CLS)))
    b3_pad = jnp.pad(b3, (0, 16 - N_CLS)).reshape(1, 16)

    deg = _degrees(col_pad, ones16, zeros16)
    h1 = _mm1(x_pad, w1_pad)
    dis16, hs1 = _scale1(deg, h1)

    p1 = _aggregate(hs1, row_pad, col_pad, zeros128, HID)
    hs2 = _layer(p1, hs1, dis16, b1.reshape(1, HID), W2)

    p2 = _aggregate(hs2, row_pad, col_pad, zeros128, HID)
    q2 = _prescale3(p2, hs2, dis16, b2.reshape(1, HID))

    p3 = _aggregate(q2, row_pad, col_pad, zeros128, HID)
    out = _final(p3, q2, dis16, w3_pad, b3_pad)
    return out[:, :N_CLS]


# speedup vs baseline: 18.0334x; 3.7394x over previous
import functools

import jax
import jax.numpy as jnp
from jax import lax
from jax.experimental import pallas as pl
from jax.experimental.pallas import tpu as pltpu
from jax.experimental.pallas import tpu_sc as plsc

N_NODES = 10000
N_EDGES = 320000
F_IN = 1433
HID = 128
N_CLS = 7

NC = 2
NS = 16
CHUNK = 128
BLOCKS_PER_CORE = N_EDGES // CHUNK // NC
BASE_BLOCKS = BLOCKS_PER_CORE // NS
EXTRA_TILES = BLOCKS_PER_CORE - BASE_BLOCKS * NS
PH0_BLOCKS = 40
PH1_BLOCKS = BASE_BLOCKS - PH0_BLOCKS
IDX_BLKS = PH0_BLOCKS
INIT_ROWS = 632
INIT_LAST = N_NODES - 15 * INIT_ROWS

K_BLK = 128
K_BLOCKS = 12
K_TAIL = F_IN - (K_BLOCKS - 1) * K_BLK


def _sc_mesh():
    return plsc.VectorSubcoreMesh(core_axis_name="c", subcore_axis_name="s")


def _init_acc(zeros_hbm, acc_sh, s):
    @pl.when(s < 15)
    def _():
        pltpu.sync_copy(zeros_hbm,
                        acc_sh.at[pl.ds(s * INIT_ROWS, INIT_ROWS)])

    @pl.when(s == 15)
    def _():
        pltpu.sync_copy(zeros_hbm.at[pl.ds(0, INIT_LAST)],
                        acc_sh.at[pl.ds(15 * INIT_ROWS, INIT_LAST)])


def _writeback(acc_sh, out_core, s):
    @pl.when(s < 15)
    def _():
        sl = pl.ds(s * INIT_ROWS, INIT_ROWS)
        pltpu.sync_copy(acc_sh.at[sl], out_core.at[sl])

    @pl.when(s == 15)
    def _():
        sl = pl.ds(15 * INIT_ROWS, INIT_LAST)
        pltpu.sync_copy(acc_sh.at[sl], out_core.at[sl])


def _agg_kernel(hs_hbm, ei_hbm, zeros_hbm, out_hbm,
                row_v, col_v, rows_a, rows_b, sem_a, sem_b, acc_sh):
    c = lax.axis_index("c")
    s = lax.axis_index("s")
    _init_acc(zeros_hbm, acc_sh, s)
    tb = (c * BLOCKS_PER_CORE + s * BASE_BLOCKS
          + jnp.minimum(s, EXTRA_TILES)) * CHUNK

    def _stage(nblk, off):
        for plane, dst in ((0, row_v), (1, col_v)):
            pltpu.sync_copy(
                ei_hbm.at[plane].at[pl.ds(tb + off * CHUNK, nblk * CHUNK)],
                dst.at[pl.ds(0, nblk * CHUNK)])

    bufs = (rows_a, rows_b)
    sems = (sem_a, sem_b)

    def _gather(g, b):
        return pltpu.make_async_copy(
            hs_hbm.at[row_v.at[pl.ds(g * CHUNK, CHUNK)]], bufs[b], sems[b])

    def _scatter(g, b):
        pltpu.sync_copy(bufs[b],
                        acc_sh.at[col_v.at[pl.ds(g * CHUNK, CHUNK)]],
                        add=True)

    def _ring(nblk):
        for b in range(2):
            _gather(b, b).start()

        @pl.loop(0, nblk // 2)
        def _(gg):
            for b in range(2):
                g = gg * 2 + b
                _gather(g, b).wait()
                _scatter(g, b)

                @pl.when(g + 2 < nblk)
                def _():
                    _gather(g + 2, b).start()

    _stage(PH0_BLOCKS, 0)
    plsc.subcore_barrier()
    _ring(PH0_BLOCKS)

    _stage(PH1_BLOCKS, PH0_BLOCKS)

    @pl.when(s < EXTRA_TILES)
    def _():
        for plane, dst in ((0, row_v), (1, col_v)):
            pltpu.sync_copy(
                ei_hbm.at[plane].at[
                    pl.ds(tb + BASE_BLOCKS * CHUNK, CHUNK)],
                dst.at[pl.ds(PH1_BLOCKS * CHUNK, CHUNK)])

    _ring(PH1_BLOCKS)

    @pl.when(s < EXTRA_TILES)
    def _():
        _gather(PH1_BLOCKS, 0).start()
        _gather(PH1_BLOCKS, 0).wait()
        _scatter(PH1_BLOCKS, 0)

    plsc.subcore_barrier()
    _writeback(acc_sh, out_hbm.at[c], s)


def _aggregate(hs, edge_index, zeros, width):
    kfn = pl.kernel(
        _agg_kernel,
        out_type=jax.ShapeDtypeStruct((NC, N_NODES, width), jnp.float32),
        mesh=_sc_mesh(),
        scratch_types=[
            pltpu.VMEM((IDX_BLKS * CHUNK,), jnp.int32),
            pltpu.VMEM((IDX_BLKS * CHUNK,), jnp.int32),
            pltpu.VMEM((CHUNK, width), jnp.float32),
            pltpu.VMEM((CHUNK, width), jnp.float32),
            pltpu.SemaphoreType.DMA,
            pltpu.SemaphoreType.DMA,
            pltpu.VMEM_SHARED((N_NODES, width), jnp.float32),
        ],
    )
    return kfn(hs, edge_index, zeros)


_BM = 1000


def _mm1_body(x_ref, w_ref, o_ref):
    k = pl.program_id(1)
    xb = x_ref[...]
    wb = w_ref[...]
    tail = k == K_BLOCKS - 1
    lane = lax.broadcasted_iota(jnp.int32, xb.shape, 1)
    xb = jnp.where(tail & (lane >= K_TAIL), 0.0, xb)
    rows = lax.broadcasted_iota(jnp.int32, wb.shape, 0)
    wb = jnp.where(tail & (rows >= K_TAIL), 0.0, wb)

    @pl.when(k == 0)
    def _():
        o_ref[...] = jnp.zeros_like(o_ref)

    o_ref[...] += jnp.dot(xb, wb, preferred_element_type=jnp.float32,
                          precision=lax.Precision.HIGHEST)


def _mm1(x, w1):
    return pl.pallas_call(
        _mm1_body,
        grid=(N_NODES // _BM, K_BLOCKS),
        in_specs=[pl.BlockSpec((_BM, K_BLK), lambda i, k: (i, k)),
                  pl.BlockSpec((K_BLK, HID), lambda i, k: (k, 0))],
        out_specs=pl.BlockSpec((_BM, HID), lambda i, k: (i, 0)),
        out_shape=jax.ShapeDtypeStruct((N_NODES, HID), jnp.float32),
    )(x, w1)


def _scale1_body(deg_ref, h_ref, dis_ref, hs_ref):
    deg = deg_ref[0] + deg_ref[1] + 1.0
    dis = lax.rsqrt(deg)
    dis_ref[...] = dis[:, :16]
    hs_ref[...] = h_ref[...] * dis


def _scale1(deg, h1):
    return pl.pallas_call(
        _scale1_body,
        grid=(N_NODES // _BM,),
        in_specs=[pl.BlockSpec((NC, _BM, HID), lambda i: (0, i, 0)),
                  pl.BlockSpec((_BM, HID), lambda i: (i, 0))],
        out_specs=[pl.BlockSpec((_BM, 16), lambda i: (i, 0)),
                   pl.BlockSpec((_BM, HID), lambda i: (i, 0))],
        out_shape=[jax.ShapeDtypeStruct((N_NODES, 16), jnp.float32),
                   jax.ShapeDtypeStruct((N_NODES, HID), jnp.float32)],
    )(deg, h1)


def _layer_body(p_ref, hs_ref, dis_ref, b_ref, w_ref, o_ref):
    dis = dis_ref[:, :1]
    agg = p_ref[0] + p_ref[1] + hs_ref[...]
    z = jnp.maximum(agg * dis + b_ref[...], 0.0)
    h = jnp.dot(z, w_ref[...], preferred_element_type=jnp.float32,
                precision=lax.Precision.HIGHEST)
    o_ref[...] = h * dis


def _layer(partials, hs_prev, dis16, b_prev, w_next):
    return pl.pallas_call(
        _layer_body,
        grid=(N_NODES // _BM,),
        in_specs=[pl.BlockSpec((NC, _BM, HID), lambda i: (0, i, 0)),
                  pl.BlockSpec((_BM, HID), lambda i: (i, 0)),
                  pl.BlockSpec((_BM, 16), lambda i: (i, 0)),
                  pl.BlockSpec((1, HID), lambda i: (0, 0)),
                  pl.BlockSpec((HID, HID), lambda i: (0, 0))],
        out_specs=pl.BlockSpec((_BM, HID), lambda i: (i, 0)),
        out_shape=jax.ShapeDtypeStruct((N_NODES, HID), jnp.float32),
    )(partials, hs_prev, dis16, b_prev, w_next)


def _prescale3_body(p_ref, hs_ref, dis_ref, b_ref, o_ref):
    dis = dis_ref[:, :1]
    agg = p_ref[0] + p_ref[1] + hs_ref[...]
    o_ref[...] = jnp.maximum(agg * dis + b_ref[...], 0.0) * dis


def _prescale3(partials, hs_prev, dis16, b_prev):
    return pl.pallas_call(
        _prescale3_body,
        grid=(N_NODES // _BM,),
        in_specs=[pl.BlockSpec((NC, _BM, HID), lambda i: (0, i, 0)),
                  pl.BlockSpec((_BM, HID), lambda i: (i, 0)),
                  pl.BlockSpec((_BM, 16), lambda i: (i, 0)),
                  pl.BlockSpec((1, HID), lambda i: (0, 0))],
        out_specs=pl.BlockSpec((_BM, HID), lambda i: (i, 0)),
        out_shape=jax.ShapeDtypeStruct((N_NODES, HID), jnp.float32),
    )(partials, hs_prev, dis16, b_prev)


def _final_body(p_ref, q_ref, dis_ref, w_ref, b_ref, o_ref):
    t = (p_ref[0] + p_ref[1] + q_ref[...]) * dis_ref[:, :1]
    h = jnp.dot(t, w_ref[...], preferred_element_type=jnp.float32,
                precision=lax.Precision.HIGHEST) + b_ref[...]
    o_ref[...] = h[:, :N_CLS]


def _final(partials, q2, dis16, w3_pad, b3_pad):
    return pl.pallas_call(
        _final_body,
        grid=(N_NODES // _BM,),
        in_specs=[pl.BlockSpec((NC, _BM, HID), lambda i: (0, i, 0)),
                  pl.BlockSpec((_BM, HID), lambda i: (i, 0)),
                  pl.BlockSpec((_BM, 16), lambda i: (i, 0)),
                  pl.BlockSpec((HID, 16), lambda i: (0, 0)),
                  pl.BlockSpec((1, 16), lambda i: (0, 0))],
        out_specs=pl.BlockSpec((_BM, N_CLS), lambda i: (i, 0)),
        out_shape=jax.ShapeDtypeStruct((N_NODES, N_CLS), jnp.float32),
    )(partials, q2, dis16, w3_pad, b3_pad)


def kernel(x, edge_index, W1, b1, W2, b2, W3, b3):
    f32 = jnp.float32
    ei = edge_index.astype(jnp.int32)

    ones_table = jnp.ones((N_NODES, HID), f32)
    zeros128 = jnp.zeros((INIT_ROWS, HID), f32)

    w3_pad = jnp.pad(W3, ((0, 0), (0, 16 - N_CLS)))
    b3_pad = jnp.pad(b3, (0, 16 - N_CLS)).reshape(1, 16)

    deg = _aggregate(ones_table, ei, zeros128, HID)
    h1 = _mm1(x, W1)
    dis16, hs1 = _scale1(deg, h1)

    p1 = _aggregate(hs1, ei, zeros128, HID)
    hs2 = _layer(p1, hs1, dis16, b1.reshape(1, HID), W2)

    p2 = _aggregate(hs2, ei, zeros128, HID)
    q2 = _prescale3(p2, hs2, dis16, b2.reshape(1, HID))

    p3 = _aggregate(q2, ei, zeros128, HID)
    return _final(p3, q2, dis16, w3_pad, b3_pad)


# speedup vs baseline: 18.4945x; 1.0256x over previous
import functools

import jax
import jax.numpy as jnp
from jax import lax
from jax.experimental import pallas as pl
from jax.experimental.pallas import tpu as pltpu
from jax.experimental.pallas import tpu_sc as plsc

N_NODES = 10000
N_EDGES = 320000
F_IN = 1433
HID = 128
N_CLS = 7

NC = 2
NS = 16
CHUNK = 128
BLOCKS_PER_CORE = N_EDGES // CHUNK // NC
BASE_BLOCKS = BLOCKS_PER_CORE // NS
EXTRA_TILES = BLOCKS_PER_CORE - BASE_BLOCKS * NS
PH0_BLOCKS = 40
PH1_BLOCKS = BASE_BLOCKS - PH0_BLOCKS
IDX_BLKS = PH0_BLOCKS
INIT_ROWS = 632
INIT_LAST = N_NODES - 15 * INIT_ROWS

K_BLK = 128
K_BLOCKS = 12
K_TAIL = F_IN - (K_BLOCKS - 1) * K_BLK


def _sc_mesh():
    return plsc.VectorSubcoreMesh(core_axis_name="c", subcore_axis_name="s")


def _init_acc(zeros_hbm, acc_sh, s):
    @pl.when(s < 15)
    def _():
        pltpu.sync_copy(zeros_hbm,
                        acc_sh.at[pl.ds(s * INIT_ROWS, INIT_ROWS)])

    @pl.when(s == 15)
    def _():
        pltpu.sync_copy(zeros_hbm.at[pl.ds(0, INIT_LAST)],
                        acc_sh.at[pl.ds(15 * INIT_ROWS, INIT_LAST)])


def _writeback(acc_sh, out_core, s):
    @pl.when(s < 15)
    def _():
        sl = pl.ds(s * INIT_ROWS, INIT_ROWS)
        pltpu.sync_copy(acc_sh.at[sl], out_core.at[sl])

    @pl.when(s == 15)
    def _():
        sl = pl.ds(15 * INIT_ROWS, INIT_LAST)
        pltpu.sync_copy(acc_sh.at[sl], out_core.at[sl])


def _deg_kernel(ei_hbm, ones_hbm, zeros_hbm, out_hbm, col_v, ones_v, sem,
                acc_sh):
    c = lax.axis_index("c")
    s = lax.axis_index("s")
    _init_acc(zeros_hbm, acc_sh, s)
    pltpu.sync_copy(ones_hbm, ones_v)
    tb = (c * BLOCKS_PER_CORE + s * BASE_BLOCKS
          + jnp.minimum(s, EXTRA_TILES)) * CHUNK
    pltpu.sync_copy(ei_hbm.at[1].at[pl.ds(tb, BASE_BLOCKS * CHUNK)],
                    col_v.at[pl.ds(0, BASE_BLOCKS * CHUNK)])

    @pl.when(s < EXTRA_TILES)
    def _():
        pltpu.sync_copy(
            ei_hbm.at[1].at[pl.ds(tb + BASE_BLOCKS * CHUNK, CHUNK)],
            col_v.at[pl.ds(BASE_BLOCKS * CHUNK, CHUNK)])

    plsc.subcore_barrier()

    def _scat(g):
        return pltpu.make_async_copy(
            ones_v, acc_sh.at[col_v.at[pl.ds(g * CHUNK, CHUNK)]], sem)

    @pl.loop(0, BASE_BLOCKS)
    def _(g):
        _scat(g).start(add=True)

    @pl.when(s < EXTRA_TILES)
    def _():
        _scat(BASE_BLOCKS).start(add=True)

    @pl.loop(0, BASE_BLOCKS)
    def _(g):
        _scat(g).wait()

    @pl.when(s < EXTRA_TILES)
    def _():
        _scat(BASE_BLOCKS).wait()

    plsc.subcore_barrier()
    _writeback(acc_sh, out_hbm.at[c], s)


def _degrees(edge_index, ones16, zeros16):
    kfn = pl.kernel(
        _deg_kernel,
        out_type=jax.ShapeDtypeStruct((NC, N_NODES, 16), jnp.float32),
        mesh=_sc_mesh(),
        scratch_types=[
            pltpu.VMEM(((BASE_BLOCKS + 1) * CHUNK,), jnp.int32),
            pltpu.VMEM((CHUNK, 16), jnp.float32),
            pltpu.SemaphoreType.DMA,
            pltpu.VMEM_SHARED((N_NODES, 16), jnp.float32),
        ],
    )
    return kfn(edge_index, ones16, zeros16)


def _agg_kernel(hs_hbm, ei_hbm, zeros_hbm, out_hbm,
                row_v, col_v, rows_a, rows_b, sem_a, sem_b, acc_sh):
    c = lax.axis_index("c")
    s = lax.axis_index("s")
    _init_acc(zeros_hbm, acc_sh, s)
    tb = (c * BLOCKS_PER_CORE + s * BASE_BLOCKS
          + jnp.minimum(s, EXTRA_TILES)) * CHUNK

    def _stage(nblk, off):
        for plane, dst in ((0, row_v), (1, col_v)):
            pltpu.sync_copy(
                ei_hbm.at[plane].at[pl.ds(tb + off * CHUNK, nblk * CHUNK)],
                dst.at[pl.ds(0, nblk * CHUNK)])

    bufs = (rows_a, rows_b)
    sems = (sem_a, sem_b)

    def _gather(g, b):
        return pltpu.make_async_copy(
            hs_hbm.at[row_v.at[pl.ds(g * CHUNK, CHUNK)]], bufs[b], sems[b])

    def _scatter(g, b):
        pltpu.sync_copy(bufs[b],
                        acc_sh.at[col_v.at[pl.ds(g * CHUNK, CHUNK)]],
                        add=True)

    def _ring(nblk):
        for b in range(2):
            _gather(b, b).start()

        @pl.loop(0, nblk // 2)
        def _(gg):
            for b in range(2):
                g = gg * 2 + b
                _gather(g, b).wait()
                _scatter(g, b)

                @pl.when(g + 2 < nblk)
                def _():
                    _gather(g + 2, b).start()

    _stage(PH0_BLOCKS, 0)
    plsc.subcore_barrier()
    _ring(PH0_BLOCKS)

    _stage(PH1_BLOCKS, PH0_BLOCKS)

    @pl.when(s < EXTRA_TILES)
    def _():
        for plane, dst in ((0, row_v), (1, col_v)):
            pltpu.sync_copy(
                ei_hbm.at[plane].at[
                    pl.ds(tb + BASE_BLOCKS * CHUNK, CHUNK)],
                dst.at[pl.ds(PH1_BLOCKS * CHUNK, CHUNK)])

    _ring(PH1_BLOCKS)

    @pl.when(s < EXTRA_TILES)
    def _():
        _gather(PH1_BLOCKS, 0).start()
        _gather(PH1_BLOCKS, 0).wait()
        _scatter(PH1_BLOCKS, 0)

    plsc.subcore_barrier()
    _writeback(acc_sh, out_hbm.at[c], s)


def _aggregate(hs, edge_index, zeros, width):
    kfn = pl.kernel(
        _agg_kernel,
        out_type=jax.ShapeDtypeStruct((NC, N_NODES, width), jnp.float32),
        mesh=_sc_mesh(),
        scratch_types=[
            pltpu.VMEM((IDX_BLKS * CHUNK,), jnp.int32),
            pltpu.VMEM((IDX_BLKS * CHUNK,), jnp.int32),
            pltpu.VMEM((CHUNK, width), jnp.float32),
            pltpu.VMEM((CHUNK, width), jnp.float32),
            pltpu.SemaphoreType.DMA,
            pltpu.SemaphoreType.DMA,
            pltpu.VMEM_SHARED((N_NODES, width), jnp.float32),
        ],
    )
    return kfn(hs, edge_index, zeros)


_BM = 1000


def _mm1_body(x_ref, w_ref, o_ref):
    k = pl.program_id(1)
    xb = x_ref[...]
    wb = w_ref[...]
    tail = k == K_BLOCKS - 1
    lane = lax.broadcasted_iota(jnp.int32, xb.shape, 1)
    xb = jnp.where(tail & (lane >= K_TAIL), 0.0, xb)
    rows = lax.broadcasted_iota(jnp.int32, wb.shape, 0)
    wb = jnp.where(tail & (rows >= K_TAIL), 0.0, wb)

    @pl.when(k == 0)
    def _():
        o_ref[...] = jnp.zeros_like(o_ref)

    o_ref[...] += jnp.dot(xb, wb, preferred_element_type=jnp.float32,
                          precision=lax.Precision.HIGHEST)


def _mm1(x, w1):
    return pl.pallas_call(
        _mm1_body,
        grid=(N_NODES // _BM, K_BLOCKS),
        in_specs=[pl.BlockSpec((_BM, K_BLK), lambda i, k: (i, k)),
                  pl.BlockSpec((K_BLK, HID), lambda i, k: (k, 0))],
        out_specs=pl.BlockSpec((_BM, HID), lambda i, k: (i, 0)),
        out_shape=jax.ShapeDtypeStruct((N_NODES, HID), jnp.float32),
    )(x, w1)


def _scale1_body(deg_ref, h_ref, dis_ref, hs_ref):
    deg = deg_ref[0] + deg_ref[1] + 1.0
    dis = lax.rsqrt(deg)
    dis_ref[...] = dis
    hs_ref[...] = h_ref[...] * dis[:, :1]


def _scale1(deg, h1):
    return pl.pallas_call(
        _scale1_body,
        grid=(N_NODES // _BM,),
        in_specs=[pl.BlockSpec((NC, _BM, 16), lambda i: (0, i, 0)),
                  pl.BlockSpec((_BM, HID), lambda i: (i, 0))],
        out_specs=[pl.BlockSpec((_BM, 16), lambda i: (i, 0)),
                   pl.BlockSpec((_BM, HID), lambda i: (i, 0))],
        out_shape=[jax.ShapeDtypeStruct((N_NODES, 16), jnp.float32),
                   jax.ShapeDtypeStruct((N_NODES, HID), jnp.float32)],
    )(deg, h1)


def _layer_body(p_ref, hs_ref, dis_ref, b_ref, w_ref, o_ref):
    dis = dis_ref[:, :1]
    agg = p_ref[0] + p_ref[1] + hs_ref[...]
    z = jnp.maximum(agg * dis + b_ref[...], 0.0)
    h = jnp.dot(z, w_ref[...], preferred_element_type=jnp.float32,
                precision=lax.Precision.HIGHEST)
    o_ref[...] = h * dis


def _layer(partials, hs_prev, dis16, b_prev, w_next):
    return pl.pallas_call(
        _layer_body,
        grid=(N_NODES // _BM,),
        in_specs=[pl.BlockSpec((NC, _BM, HID), lambda i: (0, i, 0)),
                  pl.BlockSpec((_BM, HID), lambda i: (i, 0)),
                  pl.BlockSpec((_BM, 16), lambda i: (i, 0)),
                  pl.BlockSpec((1, HID), lambda i: (0, 0)),
                  pl.BlockSpec((HID, HID), lambda i: (0, 0))],
        out_specs=pl.BlockSpec((_BM, HID), lambda i: (i, 0)),
        out_shape=jax.ShapeDtypeStruct((N_NODES, HID), jnp.float32),
    )(partials, hs_prev, dis16, b_prev, w_next)


def _prescale3_body(p_ref, hs_ref, dis_ref, b_ref, o_ref):
    dis = dis_ref[:, :1]
    agg = p_ref[0] + p_ref[1] + hs_ref[...]
    o_ref[...] = jnp.maximum(agg * dis + b_ref[...], 0.0) * dis


def _prescale3(partials, hs_prev, dis16, b_prev):
    return pl.pallas_call(
        _prescale3_body,
        grid=(N_NODES // _BM,),
        in_specs=[pl.BlockSpec((NC, _BM, HID), lambda i: (0, i, 0)),
                  pl.BlockSpec((_BM, HID), lambda i: (i, 0)),
                  pl.BlockSpec((_BM, 16), lambda i: (i, 0)),
                  pl.BlockSpec((1, HID), lambda i: (0, 0))],
        out_specs=pl.BlockSpec((_BM, HID), lambda i: (i, 0)),
        out_shape=jax.ShapeDtypeStruct((N_NODES, HID), jnp.float32),
    )(partials, hs_prev, dis16, b_prev)


def _final_body(p_ref, q_ref, dis_ref, w_ref, b_ref, o_ref):
    t = (p_ref[0] + p_ref[1] + q_ref[...]) * dis_ref[:, :1]
    h = jnp.dot(t, w_ref[...], preferred_element_type=jnp.float32,
                precision=lax.Precision.HIGHEST) + b_ref[...]
    o_ref[...] = h[:, :N_CLS]


def _final(partials, q2, dis16, w3_pad, b3_pad):
    return pl.pallas_call(
        _final_body,
        grid=(N_NODES // _BM,),
        in_specs=[pl.BlockSpec((NC, _BM, HID), lambda i: (0, i, 0)),
                  pl.BlockSpec((_BM, HID), lambda i: (i, 0)),
                  pl.BlockSpec((_BM, 16), lambda i: (i, 0)),
                  pl.BlockSpec((HID, 16), lambda i: (0, 0)),
                  pl.BlockSpec((1, 16), lambda i: (0, 0))],
        out_specs=pl.BlockSpec((_BM, N_CLS), lambda i: (i, 0)),
        out_shape=jax.ShapeDtypeStruct((N_NODES, N_CLS), jnp.float32),
    )(partials, q2, dis16, w3_pad, b3_pad)


def kernel(x, edge_index, W1, b1, W2, b2, W3, b3):
    f32 = jnp.float32
    ei = edge_index.astype(jnp.int32)

    ones16 = jnp.ones((CHUNK, 16), f32)
    zeros16 = jnp.zeros((INIT_ROWS, 16), f32)
    zeros128 = jnp.zeros((INIT_ROWS, HID), f32)

    w3_pad = jnp.pad(W3, ((0, 0), (0, 16 - N_CLS)))
    b3_pad = jnp.pad(b3, (0, 16 - N_CLS)).reshape(1, 16)

    deg = _degrees(ei, ones16, zeros16)
    h1 = _mm1(x, W1)
    dis16, hs1 = _scale1(deg, h1)

    p1 = _aggregate(hs1, ei, zeros128, HID)
    hs2 = _layer(p1, hs1, dis16, b1.reshape(1, HID), W2)

    p2 = _aggregate(hs2, ei, zeros128, HID)
    q2 = _prescale3(p2, hs2, dis16, b2.reshape(1, HID))

    p3 = _aggregate(q2, ei, zeros128, HID)
    return _final(p3, q2, dis16, w3_pad, b3_pad)
